# Initial kernel scaffold; baseline (speedup 1.0000x reference)
import jax
import jax.numpy as jnp
from jax.experimental import pallas as pl


def kernel(input_ids, wte):
    raise NotImplementedError("write your pallas kernel here")


# speedup vs baseline: 1.4732x; 1.4732x over previous
import functools

import jax
import jax.numpy as jnp
from jax import lax
from jax.experimental import pallas as pl
from jax.experimental.pallas import tpu as pltpu
from jax.experimental.pallas import tpu_sc as plsc

D_MODEL = 4096
NUM_CORES = 2
NUM_SUBCORES = 16
NUM_WORKERS = NUM_CORES * NUM_SUBCORES
TOTAL_IDS = 8192
IDS_PER_WORKER = TOTAL_IDS // NUM_WORKERS
CHUNK = 8
NUM_CHUNKS = IDS_PER_WORKER // CHUNK


def _make_emb_kernel():
    mesh = plsc.VectorSubcoreMesh(core_axis_name="c", subcore_axis_name="s")

    @functools.partial(
        pl.kernel,
        mesh=mesh,
        out_type=jax.ShapeDtypeStruct((TOTAL_IDS, D_MODEL), jnp.float32),
        scratch_types=[
            pltpu.VMEM((NUM_CHUNKS, CHUNK), jnp.int32),
            pltpu.VMEM((CHUNK, D_MODEL), jnp.float32),
            pltpu.SemaphoreType.DMA,
        ],
    )
    def emb(idx_hbm, table_hbm, out_hbm, idx_v, rows_v, sem):
        wid = lax.axis_index("s") * NUM_CORES + lax.axis_index("c")
        base = wid * IDS_PER_WORKER
        pltpu.sync_copy(idx_hbm.at[wid], idx_v)

        def body(i, carry):
            pltpu.async_copy(table_hbm.at[idx_v.at[i]], rows_v, sem).wait()
            pltpu.sync_copy(rows_v, out_hbm.at[pl.ds(base + i * CHUNK, CHUNK)])
            return carry

        lax.fori_loop(0, NUM_CHUNKS, body, 0)

    return emb


_emb = _make_emb_kernel()


def kernel(input_ids, wte):
    input_shape = input_ids.shape
    flat = input_ids.reshape(-1).astype(jnp.int32)
    idx3 = flat.reshape(NUM_WORKERS, NUM_CHUNKS, CHUNK)
    out = _emb(idx3, wte)
    return out.reshape((-1, input_shape[-1], D_MODEL))


# speedup vs baseline: 1.6689x; 1.1328x over previous
import functools

import jax
import jax.numpy as jnp
from jax import lax
from jax.experimental import pallas as pl
from jax.experimental.pallas import tpu as pltpu
from jax.experimental.pallas import tpu_sc as plsc

D_MODEL = 4096
NUM_CORES = 2
NUM_SUBCORES = 16
NUM_WORKERS = NUM_CORES * NUM_SUBCORES
TOTAL_IDS = 8192
IDS_PER_WORKER = TOTAL_IDS // NUM_WORKERS
CHUNK = 8
NUM_CHUNKS = IDS_PER_WORKER // CHUNK


NBUF = 2


def _make_emb_kernel():
    mesh = plsc.VectorSubcoreMesh(core_axis_name="c", subcore_axis_name="s")

    @functools.partial(
        pl.kernel,
        mesh=mesh,
        out_type=jax.ShapeDtypeStruct((TOTAL_IDS, D_MODEL), jnp.float32),
        scratch_types=[
            pltpu.VMEM((NUM_CHUNKS, CHUNK), jnp.int32),
            pltpu.VMEM((NBUF, CHUNK, D_MODEL), jnp.float32),
            pltpu.SemaphoreType.DMA,
            pltpu.SemaphoreType.DMA,
            pltpu.SemaphoreType.DMA,
            pltpu.SemaphoreType.DMA,
        ],
    )
    def emb(idx_hbm, table_hbm, out_hbm, idx_v, rows_v, g0, g1, s0, s1):
        gsems = (g0, g1)
        ssems = (s0, s1)
        wid = lax.axis_index("s") * NUM_CORES + lax.axis_index("c")
        base = wid * IDS_PER_WORKER
        pltpu.sync_copy(idx_hbm.at[wid], idx_v)

        def start_gather(i, b):
            pltpu.async_copy(table_hbm.at[idx_v.at[i]], rows_v.at[b], gsems[b])

        def wait_gather(i, b):
            pltpu.make_async_copy(
                table_hbm.at[idx_v.at[i]], rows_v.at[b], gsems[b]
            ).wait()

        def start_store(i, b):
            pltpu.async_copy(
                rows_v.at[b], out_hbm.at[pl.ds(base + i * CHUNK, CHUNK)], ssems[b]
            )

        def wait_store(i, b):
            pltpu.make_async_copy(
                rows_v.at[b], out_hbm.at[pl.ds(base + i * CHUNK, CHUNK)], ssems[b]
            ).wait()

        for b in range(NBUF):
            start_gather(b, b)

        def group(g, carry):
            for b in range(NBUF):
                i = g * NBUF + b
                wait_gather(i, b)
                start_store(i, b)
            for b in range(NBUF):
                i = g * NBUF + b
                wait_store(i, b)

                @pl.when(i + NBUF < NUM_CHUNKS)
                def _():
                    start_gather(i + NBUF, b)

            return carry

        lax.fori_loop(0, NUM_CHUNKS // NBUF, group, 0)

    return emb


_emb = _make_emb_kernel()


def kernel(input_ids, wte):
    input_shape = input_ids.shape
    flat = input_ids.reshape(-1).astype(jnp.int32)
    idx3 = flat.reshape(NUM_WORKERS, NUM_CHUNKS, CHUNK)
    out = _emb(idx3, wte)
    return out.reshape((-1, input_shape[-1], D_MODEL))


# speedup vs baseline: 1.7161x; 1.0283x over previous
import functools

import jax
import jax.numpy as jnp
from jax import lax
from jax.experimental import pallas as pl
from jax.experimental.pallas import tpu as pltpu
from jax.experimental.pallas import tpu_sc as plsc

D_MODEL = 4096
NUM_CORES = 2
NUM_SUBCORES = 16
NUM_WORKERS = NUM_CORES * NUM_SUBCORES
TOTAL_IDS = 8192
IDS_PER_WORKER = TOTAL_IDS // NUM_WORKERS
CHUNK = 4
NUM_CHUNKS = IDS_PER_WORKER // CHUNK


NBUF = 4


def _make_emb_kernel():
    mesh = plsc.VectorSubcoreMesh(core_axis_name="c", subcore_axis_name="s")

    @functools.partial(
        pl.kernel,
        mesh=mesh,
        out_type=jax.ShapeDtypeStruct((TOTAL_IDS, D_MODEL), jnp.float32),
        scratch_types=[
            pltpu.VMEM((NUM_CHUNKS, CHUNK), jnp.int32),
            pltpu.VMEM((NBUF, CHUNK, D_MODEL), jnp.float32),
        ] + [pltpu.SemaphoreType.DMA] * (2 * NBUF),
    )
    def emb(idx_hbm, table_hbm, out_hbm, idx_v, rows_v, *sems):
        gsems = sems[:NBUF]
        ssems = sems[NBUF:]
        wid = lax.axis_index("s") * NUM_CORES + lax.axis_index("c")
        base = wid * IDS_PER_WORKER
        pltpu.sync_copy(idx_hbm.at[wid], idx_v)

        def start_gather(i, b):
            pltpu.async_copy(table_hbm.at[idx_v.at[i]], rows_v.at[b], gsems[b])

        def wait_gather(i, b):
            pltpu.make_async_copy(
                table_hbm.at[idx_v.at[i]], rows_v.at[b], gsems[b]
            ).wait()

        def start_store(i, b):
            pltpu.async_copy(
                rows_v.at[b], out_hbm.at[pl.ds(base + i * CHUNK, CHUNK)], ssems[b]
            )

        def wait_store(i, b):
            pltpu.make_async_copy(
                rows_v.at[b], out_hbm.at[pl.ds(base + i * CHUNK, CHUNK)], ssems[b]
            ).wait()

        for b in range(NBUF):
            start_gather(b, b)

        def group(g, carry):
            for b in range(NBUF):
                i = g * NBUF + b
                wait_gather(i, b)
                start_store(i, b)
            for b in range(NBUF):
                i = g * NBUF + b
                wait_store(i, b)

                @pl.when(i + NBUF < NUM_CHUNKS)
                def _():
                    start_gather(i + NBUF, b)

            return carry

        lax.fori_loop(0, NUM_CHUNKS // NBUF, group, 0)

    return emb


_emb = _make_emb_kernel()


def kernel(input_ids, wte):
    input_shape = input_ids.shape
    flat = input_ids.reshape(-1).astype(jnp.int32)
    idx3 = flat.reshape(NUM_WORKERS, NUM_CHUNKS, CHUNK)
    out = _emb(idx3, wte)
    return out.reshape((-1, input_shape[-1], D_MODEL))
